# Initial kernel scaffold; baseline (speedup 1.0000x reference)
import jax
import jax.numpy as jnp
from jax.experimental import pallas as pl


def kernel(xyz, features, W1, b1, W2, b2, W3, b3):
    raise NotImplementedError("write your pallas kernel here")


# speedup vs baseline: 7.8995x; 7.8995x over previous
import functools

import jax
import jax.numpy as jnp
from jax import lax
from jax.experimental import pallas as pl
from jax.experimental.pallas import tpu as pltpu
from jax.experimental.pallas import tpu_sc as plsc

NPOINT = 1024
RADIUS = 0.2
NSAMPLE = 32
N = 16384
B = 4
CFEAT = 16
CIN = 3 + CFEAT
CPAD = 128
BIG_I32 = 1 << 30


def _fps_body(xr_ref, idx_out_ref, cxyz_out_ref):
    x = xr_ref[0, 0]
    y = xr_ref[0, 1]
    z = xr_ref[0, 2]
    row = lax.broadcasted_iota(jnp.int32, (128, 128), 0)
    col = lax.broadcasted_iota(jnp.int32, (128, 128), 1)
    flat = row * 128 + col
    prow = lax.broadcasted_iota(jnp.int32, (8, 128), 0)
    pcol = lax.broadcasted_iota(jnp.int32, (8, 128), 1)
    pflat = prow * 128 + pcol

    def body(i, carry):
        dists, far, aidx, ax, ay, az = carry
        aidx = jnp.where(pflat == i, far, aidx)
        onehot = flat == far
        cx = jnp.sum(jnp.where(onehot, x, 0.0))
        cy = jnp.sum(jnp.where(onehot, y, 0.0))
        cz = jnp.sum(jnp.where(onehot, z, 0.0))
        ax = jnp.where(pflat == i, cx, ax)
        ay = jnp.where(pflat == i, cy, ay)
        az = jnp.where(pflat == i, cz, az)
        d = (x - cx) ** 2 + (y - cy) ** 2 + (z - cz) ** 2
        dists = jnp.minimum(dists, d)
        m = jnp.max(dists)
        far = jnp.min(jnp.where(dists == m, flat, BIG_I32))
        return dists, far, aidx, ax, ay, az

    dists0 = jnp.full((128, 128), 1e10, dtype=jnp.float32)
    zi = jnp.zeros((8, 128), dtype=jnp.int32)
    zf = jnp.zeros((8, 128), dtype=jnp.float32)
    _, _, aidx, ax, ay, az = lax.fori_loop(
        0, NPOINT, body, (dists0, jnp.int32(0), zi, zf, zf, zf))
    idx_out_ref[0] = aidx
    cxyz_out_ref[0, 0] = ax
    cxyz_out_ref[0, 1] = ay
    cxyz_out_ref[0, 2] = az


def _run_fps(xr):
    return pl.pallas_call(
        _fps_body,
        grid=(B,),
        in_specs=[pl.BlockSpec((1, 3, 128, 128), lambda b: (b, 0, 0, 0))],
        out_specs=[
            pl.BlockSpec((1, 8, 128), lambda b: (b, 0, 0)),
            pl.BlockSpec((1, 3, 8, 128), lambda b: (b, 0, 0, 0)),
        ],
        out_shape=[
            jax.ShapeDtypeStruct((B, 8, 128), jnp.int32),
            jax.ShapeDtypeStruct((B, 3, 8, 128), jnp.float32),
        ],
    )(xr)


PTILE = 64


def _ballq_body(xt_ref, cents_ref, out_ref):
    xs = xt_ref[0]
    xx = xs[0:1, :]
    yy = xs[1:2, :]
    zz = xs[2:3, :]
    sx = xx * xx + yy * yy + zz * zz
    c = cents_ref[0]
    cx = c[:, 0:1]
    cy = c[:, 1:2]
    cz = c[:, 2:3]
    sc = cx * cx + cy * cy + cz * cz
    bf = lambda a: a.astype(jnp.bfloat16).astype(jnp.float32)
    dot = bf(cx) * bf(xx) + bf(cy) * bf(yy) + bf(cz) * bf(zz)
    d2 = sc + sx - 2.0 * dot
    colj = lax.broadcasted_iota(jnp.int32, (PTILE, N), 1)
    v0 = jnp.where(d2 <= RADIUS * RADIUS, colj, N)
    col32 = lax.broadcasted_iota(jnp.int32, (PTILE, NSAMPLE), 1)

    def body(s, carry):
        v, acc = carry
        m = jnp.min(v, axis=1, keepdims=True)
        rec = jnp.minimum(m, N)
        acc = jnp.where(col32 == s, rec, acc)
        v = jnp.where(v == m, BIG_I32, v)
        return v, acc

    acc0 = jnp.zeros((PTILE, NSAMPLE), dtype=jnp.int32)
    _, acc = lax.fori_loop(0, NSAMPLE, body, (v0, acc0))
    first = acc[:, 0:1]
    first = jnp.where(first == N, 0, first)
    acc = jnp.where(acc == N, first, acc)
    out_ref[0] = acc


def _run_ballq(xt, cents):
    return pl.pallas_call(
        _ballq_body,
        grid=(B, NPOINT // PTILE),
        in_specs=[
            pl.BlockSpec((1, 3, N), lambda b, p: (b, 0, 0)),
            pl.BlockSpec((1, PTILE, 3), lambda b, p: (b, p, 0)),
        ],
        out_specs=pl.BlockSpec((1, PTILE, NSAMPLE), lambda b, p: (b, p, 0)),
        out_shape=jax.ShapeDtypeStruct((B, NPOINT, NSAMPLE), jnp.int32),
    )(xt, cents)


GROWS = B * NPOINT * NSAMPLE
GCHUNK = 512


def _make_sc_gather():
    info = plsc.get_sparse_core_info()
    nw = info.num_cores * info.num_subcores
    b_per_w = GROWS // nw
    nchunks = b_per_w // GCHUNK
    mesh = plsc.VectorSubcoreMesh(core_axis_name="c", subcore_axis_name="s")

    @functools.partial(
        pl.kernel, mesh=mesh,
        out_type=jax.ShapeDtypeStruct((GROWS, CPAD), jnp.float32),
        scratch_types=[
            pltpu.VMEM((GCHUNK,), jnp.int32),
            pltpu.VMEM((GCHUNK, CPAD), jnp.float32),
            pltpu.SemaphoreType.DMA,
        ],
    )
    def sc_gather(table_hbm, idx_hbm, out_hbm, idx_v, rows_v, sem):
        wid = lax.axis_index("s") * info.num_cores + lax.axis_index("c")
        base = wid * b_per_w
        for k in range(nchunks):
            off = base + k * GCHUNK
            pltpu.sync_copy(idx_hbm.at[pl.ds(off, GCHUNK)], idx_v)
            pltpu.async_copy(table_hbm.at[idx_v], rows_v, sem).wait()
            pltpu.sync_copy(rows_v, out_hbm.at[pl.ds(off, GCHUNK)])

    return sc_gather


GT = 128
RT = GT * NSAMPLE


def _mlp_body(g_ref, c_ref, w1_ref, b1_ref, w2_ref, b2_ref, w3_ref, b3_ref,
              out_ref):
    g = g_ref[...]
    c = c_ref[...]
    w1 = w1_ref[...]
    cx = c[:, 0:1]
    cy = c[:, 1:2]
    cz = c[:, 2:3]
    base = cx * w1[0:1, :] + cy * w1[1:2, :] + cz * w1[2:3, :]
    base = jnp.broadcast_to(base[:, None, :], (GT, NSAMPLE, 32))
    base = base.reshape(RT, 32)
    h = jnp.dot(g, w1, preferred_element_type=jnp.float32)
    h = jnp.maximum(h + b1_ref[...] - base, 0.0)
    h = jnp.dot(h, w2_ref[...], preferred_element_type=jnp.float32)
    h = jnp.maximum(h + b2_ref[...], 0.0)
    h = jnp.dot(h, w3_ref[...], preferred_element_type=jnp.float32)
    h = jnp.maximum(h + b3_ref[...], 0.0)
    out_ref[...] = jnp.max(h.reshape(GT, NSAMPLE, 64), axis=1)


def _run_mlp(g, cflat, w1p, b1r, w2, b2r, w3, b3r):
    ngroups = B * NPOINT
    return pl.pallas_call(
        _mlp_body,
        grid=(ngroups // GT,),
        in_specs=[
            pl.BlockSpec((RT, CPAD), lambda i: (i, 0)),
            pl.BlockSpec((GT, 3), lambda i: (i, 0)),
            pl.BlockSpec((CPAD, 32), lambda i: (0, 0)),
            pl.BlockSpec((1, 32), lambda i: (0, 0)),
            pl.BlockSpec((32, 32), lambda i: (0, 0)),
            pl.BlockSpec((1, 32), lambda i: (0, 0)),
            pl.BlockSpec((32, 64), lambda i: (0, 0)),
            pl.BlockSpec((1, 64), lambda i: (0, 0)),
        ],
        out_specs=pl.BlockSpec((GT, 64), lambda i: (i, 0)),
        out_shape=jax.ShapeDtypeStruct((ngroups, 64), jnp.float32),
    )(g, cflat, w1p, b1r, w2, b2r, w3, b3r)


def kernel(xyz, features, W1, b1, W2, b2, W3, b3):
    xt = xyz.transpose(0, 2, 1)
    xr = xt.reshape(B, 3, 128, 128)

    fps_i, cxyz = _run_fps(xr)
    fps_idx = fps_i.reshape(B, NPOINT)
    new_xyz = cxyz.reshape(B, 3, NPOINT).transpose(0, 2, 1)

    idx = _run_ballq(xt, new_xyz)

    table = jnp.concatenate(
        [xyz, features,
         jnp.zeros((B, N, CPAD - CIN), dtype=jnp.float32)], axis=-1)
    table = table.reshape(B * N, CPAD)
    offs = (jnp.arange(B, dtype=jnp.int32) * N)[:, None, None]
    idx_flat = (idx + offs).reshape(GROWS)
    gathered = _make_sc_gather()(table, idx_flat)

    w1p = jnp.zeros((CPAD, 32), jnp.float32).at[:CIN].set(W1)
    pooled = _run_mlp(gathered, new_xyz.reshape(B * NPOINT, 3), w1p,
                      b1.reshape(1, 32), W2, b2.reshape(1, 32),
                      W3, b3.reshape(1, 64))
    new_features = pooled.reshape(B, NPOINT, 64).transpose(0, 2, 1)
    return new_xyz, new_features, fps_idx


# speedup vs baseline: 10.5562x; 1.3363x over previous
import functools

import jax
import jax.numpy as jnp
from jax import lax
from jax.experimental import pallas as pl
from jax.experimental.pallas import tpu as pltpu
from jax.experimental.pallas import tpu_sc as plsc

NPOINT = 1024
RADIUS = 0.2
NSAMPLE = 32
N = 16384
B = 4
CFEAT = 16
CIN = 3 + CFEAT
CPAD = 128
BIG_I32 = 1 << 30


def _fps_body(xr_ref, idx_out_ref, cx_out_ref, cy_out_ref, cz_out_ref):
    xs = xr_ref[...]
    x = xs[:, 0]
    y = xs[:, 1]
    z = xs[:, 2]
    row = lax.broadcasted_iota(jnp.int32, (B, 128, 128), 1)
    col = lax.broadcasted_iota(jnp.int32, (B, 128, 128), 2)
    flat = row * 128 + col
    prow = lax.broadcasted_iota(jnp.int32, (B, 8, 128), 1)
    pcol = lax.broadcasted_iota(jnp.int32, (B, 8, 128), 2)
    pos = prow * 128 + pcol

    def body(i, carry):
        dists, far, aidx, ax, ay, az = carry
        sel = pos == i
        aidx = jnp.where(sel, far, aidx)
        onehot = flat == far
        cx = jnp.sum(jnp.where(onehot, x, 0.0), axis=(1, 2), keepdims=True)
        cy = jnp.sum(jnp.where(onehot, y, 0.0), axis=(1, 2), keepdims=True)
        cz = jnp.sum(jnp.where(onehot, z, 0.0), axis=(1, 2), keepdims=True)
        ax = jnp.where(sel, cx, ax)
        ay = jnp.where(sel, cy, ay)
        az = jnp.where(sel, cz, az)
        d = (x - cx) ** 2 + (y - cy) ** 2 + (z - cz) ** 2
        dists = jnp.minimum(dists, d)
        m = jnp.max(dists, axis=(1, 2), keepdims=True)
        far = jnp.min(jnp.where(dists == m, flat, BIG_I32),
                      axis=(1, 2), keepdims=True)
        return dists, far, aidx, ax, ay, az

    dists0 = jnp.full((B, 128, 128), 1e10, dtype=jnp.float32)
    zi = jnp.zeros((B, 8, 128), dtype=jnp.int32)
    zf = jnp.zeros((B, 8, 128), dtype=jnp.float32)
    far0 = jnp.zeros((B, 1, 1), dtype=jnp.int32)
    _, _, aidx, ax, ay, az = lax.fori_loop(
        0, NPOINT, body, (dists0, far0, zi, zf, zf, zf))
    idx_out_ref[...] = aidx
    cx_out_ref[...] = ax
    cy_out_ref[...] = ay
    cz_out_ref[...] = az


def _run_fps(xr):
    sds = jax.ShapeDtypeStruct((B, 8, 128), jnp.float32)
    return pl.pallas_call(
        _fps_body,
        out_shape=[jax.ShapeDtypeStruct((B, 8, 128), jnp.int32),
                   sds, sds, sds],
    )(xr)


PTILE = 64


def _ballq_body(xt_ref, cents_ref, out_ref):
    xs = xt_ref[0]
    xx = xs[0:1, :]
    yy = xs[1:2, :]
    zz = xs[2:3, :]
    sx = xx * xx + yy * yy + zz * zz
    c = cents_ref[0]
    cx = c[:, 0:1]
    cy = c[:, 1:2]
    cz = c[:, 2:3]
    sc = cx * cx + cy * cy + cz * cz
    bf = lambda a: a.astype(jnp.bfloat16).astype(jnp.float32)
    dot = bf(cx) * bf(xx) + bf(cy) * bf(yy) + bf(cz) * bf(zz)
    d2 = sc + sx - 2.0 * dot
    colj = lax.broadcasted_iota(jnp.int32, (PTILE, N), 1)
    v0 = jnp.where(d2 <= RADIUS * RADIUS, colj, N)
    col32 = lax.broadcasted_iota(jnp.int32, (PTILE, NSAMPLE), 1)

    def body(s, carry):
        v, acc = carry
        m = jnp.min(v, axis=1, keepdims=True)
        rec = jnp.minimum(m, N)
        acc = jnp.where(col32 == s, rec, acc)
        v = jnp.where(v == m, BIG_I32, v)
        return v, acc

    acc0 = jnp.zeros((PTILE, NSAMPLE), dtype=jnp.int32)
    _, acc = lax.fori_loop(0, NSAMPLE, body, (v0, acc0))
    first = acc[:, 0:1]
    first = jnp.where(first == N, 0, first)
    acc = jnp.where(acc == N, first, acc)
    out_ref[0] = acc


def _run_ballq(xt, cents):
    return pl.pallas_call(
        _ballq_body,
        grid=(B, NPOINT // PTILE),
        in_specs=[
            pl.BlockSpec((1, 3, N), lambda b, p: (b, 0, 0)),
            pl.BlockSpec((1, PTILE, 3), lambda b, p: (b, p, 0)),
        ],
        out_specs=pl.BlockSpec((1, PTILE, NSAMPLE), lambda b, p: (b, p, 0)),
        out_shape=jax.ShapeDtypeStruct((B, NPOINT, NSAMPLE), jnp.int32),
    )(xt, cents)


GROWS = B * NPOINT * NSAMPLE
GCHUNK = 512


def _make_sc_gather():
    info = plsc.get_sparse_core_info()
    nw = info.num_cores * info.num_subcores
    b_per_w = GROWS // nw
    nchunks = b_per_w // GCHUNK
    mesh = plsc.VectorSubcoreMesh(core_axis_name="c", subcore_axis_name="s")

    @functools.partial(
        pl.kernel, mesh=mesh,
        out_type=jax.ShapeDtypeStruct((GROWS, CPAD), jnp.float32),
        scratch_types=[
            pltpu.VMEM((GCHUNK,), jnp.int32),
            pltpu.VMEM((GCHUNK, CPAD), jnp.float32),
            pltpu.SemaphoreType.DMA,
        ],
    )
    def sc_gather(table_hbm, idx_hbm, out_hbm, idx_v, rows_v, sem):
        wid = lax.axis_index("s") * info.num_cores + lax.axis_index("c")
        base = wid * b_per_w
        for k in range(nchunks):
            off = base + k * GCHUNK
            pltpu.sync_copy(idx_hbm.at[pl.ds(off, GCHUNK)], idx_v)
            pltpu.async_copy(table_hbm.at[idx_v], rows_v, sem).wait()
            pltpu.sync_copy(rows_v, out_hbm.at[pl.ds(off, GCHUNK)])

    return sc_gather


GT = 128
RT = GT * NSAMPLE


def _mlp_body(g_ref, c_ref, w1_ref, b1_ref, w2_ref, b2_ref, w3_ref, b3_ref,
              out_ref):
    g = g_ref[...]
    c = c_ref[...]
    w1 = w1_ref[...]
    cx = c[:, 0:1]
    cy = c[:, 1:2]
    cz = c[:, 2:3]
    base = cx * w1[0:1, :] + cy * w1[1:2, :] + cz * w1[2:3, :]
    base = jnp.broadcast_to(base[:, None, :], (GT, NSAMPLE, 32))
    base = base.reshape(RT, 32)
    h = jnp.dot(g, w1, preferred_element_type=jnp.float32)
    h = jnp.maximum(h + b1_ref[...] - base, 0.0)
    h = jnp.dot(h, w2_ref[...], preferred_element_type=jnp.float32)
    h = jnp.maximum(h + b2_ref[...], 0.0)
    h = jnp.dot(h, w3_ref[...], preferred_element_type=jnp.float32)
    h = jnp.maximum(h + b3_ref[...], 0.0)
    out_ref[...] = jnp.max(h.reshape(GT, NSAMPLE, 64), axis=1)


def _run_mlp(g, cflat, w1p, b1r, w2, b2r, w3, b3r):
    ngroups = B * NPOINT
    return pl.pallas_call(
        _mlp_body,
        grid=(ngroups // GT,),
        in_specs=[
            pl.BlockSpec((RT, CPAD), lambda i: (i, 0)),
            pl.BlockSpec((GT, 3), lambda i: (i, 0)),
            pl.BlockSpec((CPAD, 32), lambda i: (0, 0)),
            pl.BlockSpec((1, 32), lambda i: (0, 0)),
            pl.BlockSpec((32, 32), lambda i: (0, 0)),
            pl.BlockSpec((1, 32), lambda i: (0, 0)),
            pl.BlockSpec((32, 64), lambda i: (0, 0)),
            pl.BlockSpec((1, 64), lambda i: (0, 0)),
        ],
        out_specs=pl.BlockSpec((GT, 64), lambda i: (i, 0)),
        out_shape=jax.ShapeDtypeStruct((ngroups, 64), jnp.float32),
    )(g, cflat, w1p, b1r, w2, b2r, w3, b3r)


def kernel(xyz, features, W1, b1, W2, b2, W3, b3):
    xt = xyz.transpose(0, 2, 1)
    xr = xt.reshape(B, 3, 128, 128)

    fps_i, ax, ay, az = _run_fps(xr)
    fps_idx = fps_i.reshape(B, NPOINT)
    new_xyz = jnp.stack([ax.reshape(B, NPOINT), ay.reshape(B, NPOINT),
                         az.reshape(B, NPOINT)], axis=-1)

    idx = _run_ballq(xt, new_xyz)

    table = jnp.concatenate(
        [xyz, features,
         jnp.zeros((B, N, CPAD - CIN), dtype=jnp.float32)], axis=-1)
    table = table.reshape(B * N, CPAD)
    offs = (jnp.arange(B, dtype=jnp.int32) * N)[:, None, None]
    idx_flat = (idx + offs).reshape(GROWS)
    gathered = _make_sc_gather()(table, idx_flat)

    w1p = jnp.zeros((CPAD, 32), jnp.float32).at[:CIN].set(W1)
    pooled = _run_mlp(gathered, new_xyz.reshape(B * NPOINT, 3), w1p,
                      b1.reshape(1, 32), W2, b2.reshape(1, 32),
                      W3, b3.reshape(1, 64))
    new_features = pooled.reshape(B, NPOINT, 64).transpose(0, 2, 1)
    return new_xyz, new_features, fps_idx


# speedup vs baseline: 11.0469x; 1.0465x over previous
import functools

import jax
import jax.numpy as jnp
from jax import lax
from jax.experimental import pallas as pl
from jax.experimental.pallas import tpu as pltpu
from jax.experimental.pallas import tpu_sc as plsc

NPOINT = 1024
RADIUS = 0.2
NSAMPLE = 32
N = 16384
B = 4
CFEAT = 16
CIN = 3 + CFEAT
CPAD = 128
BIG_I32 = 1 << 30


def _fps_body(xr_ref, idx_out_ref, cx_out_ref, cy_out_ref, cz_out_ref):
    xs = xr_ref[...]
    x = xs[:, 0]
    y = xs[:, 1]
    z = xs[:, 2]
    row = lax.broadcasted_iota(jnp.int32, (B, 128, 128), 1)
    col = lax.broadcasted_iota(jnp.int32, (B, 128, 128), 2)
    flat = row * 128 + col
    prow = lax.broadcasted_iota(jnp.int32, (B, 8, 128), 1)
    pcol = lax.broadcasted_iota(jnp.int32, (B, 8, 128), 2)
    pos = prow * 128 + pcol

    def body(i, carry):
        dists, far, aidx, ax, ay, az = carry
        sel = pos == i
        aidx = jnp.where(sel, far, aidx)
        onehot = flat == far
        cx = jnp.sum(jnp.where(onehot, x, 0.0), axis=(1, 2), keepdims=True)
        cy = jnp.sum(jnp.where(onehot, y, 0.0), axis=(1, 2), keepdims=True)
        cz = jnp.sum(jnp.where(onehot, z, 0.0), axis=(1, 2), keepdims=True)
        ax = jnp.where(sel, cx, ax)
        ay = jnp.where(sel, cy, ay)
        az = jnp.where(sel, cz, az)
        d = (x - cx) ** 2 + (y - cy) ** 2 + (z - cz) ** 2
        dists = jnp.minimum(dists, d)
        m = jnp.max(dists, axis=(1, 2), keepdims=True)
        far = jnp.min(jnp.where(dists == m, flat, BIG_I32),
                      axis=(1, 2), keepdims=True)
        return dists, far, aidx, ax, ay, az

    dists0 = jnp.full((B, 128, 128), 1e10, dtype=jnp.float32)
    zi = jnp.zeros((B, 8, 128), dtype=jnp.int32)
    zf = jnp.zeros((B, 8, 128), dtype=jnp.float32)
    far0 = jnp.zeros((B, 1, 1), dtype=jnp.int32)
    _, _, aidx, ax, ay, az = lax.fori_loop(
        0, NPOINT, body, (dists0, far0, zi, zf, zf, zf))
    idx_out_ref[...] = aidx
    cx_out_ref[...] = ax
    cy_out_ref[...] = ay
    cz_out_ref[...] = az


def _run_fps(xr):
    sds = jax.ShapeDtypeStruct((B, 8, 128), jnp.float32)
    return pl.pallas_call(
        _fps_body,
        out_shape=[jax.ShapeDtypeStruct((B, 8, 128), jnp.int32),
                   sds, sds, sds],
    )(xr)


PTILE = 128


def _ballq_body(xt_ref, cents_ref, out_ref):
    xs = xt_ref[0]
    xx = xs[0:1, :]
    yy = xs[1:2, :]
    zz = xs[2:3, :]
    sx = xx * xx + yy * yy + zz * zz
    c = cents_ref[0]
    cx = c[:, 0:1]
    cy = c[:, 1:2]
    cz = c[:, 2:3]
    sc = cx * cx + cy * cy + cz * cz
    bf = lambda a: a.astype(jnp.bfloat16).astype(jnp.float32)
    dot = bf(cx) * bf(xx) + bf(cy) * bf(yy) + bf(cz) * bf(zz)
    d2 = sc + sx - 2.0 * dot
    colj = lax.broadcasted_iota(jnp.int32, (PTILE, N), 1)
    v0 = jnp.where(d2 <= RADIUS * RADIUS, colj, N)
    col32 = lax.broadcasted_iota(jnp.int32, (PTILE, NSAMPLE), 1)

    def body(s, carry):
        v, acc = carry
        m = jnp.min(v, axis=1, keepdims=True)
        rec = jnp.minimum(m, N)
        acc = jnp.where(col32 == s, rec, acc)
        v = jnp.where(v == m, BIG_I32, v)
        return v, acc

    acc0 = jnp.zeros((PTILE, NSAMPLE), dtype=jnp.int32)
    _, acc = lax.fori_loop(0, NSAMPLE, body, (v0, acc0))
    first = acc[:, 0:1]
    first = jnp.where(first == N, 0, first)
    acc = jnp.where(acc == N, first, acc)
    out_ref[0] = acc


def _run_ballq(xt, cents):
    return pl.pallas_call(
        _ballq_body,
        grid=(B, NPOINT // PTILE),
        in_specs=[
            pl.BlockSpec((1, 3, N), lambda b, p: (b, 0, 0)),
            pl.BlockSpec((1, PTILE, 3), lambda b, p: (b, p, 0)),
        ],
        out_specs=pl.BlockSpec((1, PTILE, NSAMPLE), lambda b, p: (b, p, 0)),
        out_shape=jax.ShapeDtypeStruct((B, NPOINT, NSAMPLE), jnp.int32),
    )(xt, cents)


GROWS = B * NPOINT * NSAMPLE
GCHUNK = 512


def _make_sc_gather():
    info = plsc.get_sparse_core_info()
    nw = info.num_cores * info.num_subcores
    b_per_w = GROWS // nw
    nchunks = b_per_w // GCHUNK
    mesh = plsc.VectorSubcoreMesh(core_axis_name="c", subcore_axis_name="s")

    @functools.partial(
        pl.kernel, mesh=mesh,
        out_type=jax.ShapeDtypeStruct((GROWS, CPAD), jnp.float32),
        scratch_types=[
            pltpu.VMEM((GCHUNK,), jnp.int32),
            pltpu.VMEM((GCHUNK, CPAD), jnp.float32),
            pltpu.SemaphoreType.DMA,
        ],
    )
    def sc_gather(table_hbm, idx_hbm, out_hbm, idx_v, rows_v, sem):
        wid = lax.axis_index("s") * info.num_cores + lax.axis_index("c")
        base = wid * b_per_w
        for k in range(nchunks):
            off = base + k * GCHUNK
            pltpu.sync_copy(idx_hbm.at[pl.ds(off, GCHUNK)], idx_v)
            pltpu.async_copy(table_hbm.at[idx_v], rows_v, sem).wait()
            pltpu.sync_copy(rows_v, out_hbm.at[pl.ds(off, GCHUNK)])

    return sc_gather


GT = 128
RT = GT * NSAMPLE


def _mlp_body(g_ref, c_ref, w1_ref, b1_ref, w2_ref, b2_ref, w3_ref, b3_ref,
              out_ref):
    g = g_ref[...]
    c = c_ref[...]
    w1 = w1_ref[...]
    cx = c[:, 0:1]
    cy = c[:, 1:2]
    cz = c[:, 2:3]
    base = cx * w1[0:1, :] + cy * w1[1:2, :] + cz * w1[2:3, :]
    base = jnp.broadcast_to(base[:, None, :], (GT, NSAMPLE, 32))
    base = base.reshape(RT, 32)
    h = jnp.dot(g, w1, preferred_element_type=jnp.float32)
    h = jnp.maximum(h + b1_ref[...] - base, 0.0)
    h = jnp.dot(h, w2_ref[...], preferred_element_type=jnp.float32)
    h = jnp.maximum(h + b2_ref[...], 0.0)
    h = jnp.dot(h, w3_ref[...], preferred_element_type=jnp.float32)
    h = jnp.maximum(h + b3_ref[...], 0.0)
    out_ref[...] = jnp.max(h.reshape(GT, NSAMPLE, 64), axis=1)


def _run_mlp(g, cflat, w1p, b1r, w2, b2r, w3, b3r):
    ngroups = B * NPOINT
    return pl.pallas_call(
        _mlp_body,
        grid=(ngroups // GT,),
        in_specs=[
            pl.BlockSpec((RT, CPAD), lambda i: (i, 0)),
            pl.BlockSpec((GT, 3), lambda i: (i, 0)),
            pl.BlockSpec((CPAD, 32), lambda i: (0, 0)),
            pl.BlockSpec((1, 32), lambda i: (0, 0)),
            pl.BlockSpec((32, 32), lambda i: (0, 0)),
            pl.BlockSpec((1, 32), lambda i: (0, 0)),
            pl.BlockSpec((32, 64), lambda i: (0, 0)),
            pl.BlockSpec((1, 64), lambda i: (0, 0)),
        ],
        out_specs=pl.BlockSpec((GT, 64), lambda i: (i, 0)),
        out_shape=jax.ShapeDtypeStruct((ngroups, 64), jnp.float32),
    )(g, cflat, w1p, b1r, w2, b2r, w3, b3r)


def kernel(xyz, features, W1, b1, W2, b2, W3, b3):
    xt = xyz.transpose(0, 2, 1)
    xr = xt.reshape(B, 3, 128, 128)

    fps_i, ax, ay, az = _run_fps(xr)
    fps_idx = fps_i.reshape(B, NPOINT)
    new_xyz = jnp.stack([ax.reshape(B, NPOINT), ay.reshape(B, NPOINT),
                         az.reshape(B, NPOINT)], axis=-1)

    idx = _run_ballq(xt, new_xyz)

    table = jnp.concatenate(
        [xyz, features,
         jnp.zeros((B, N, CPAD - CIN), dtype=jnp.float32)], axis=-1)
    table = table.reshape(B * N, CPAD)
    offs = (jnp.arange(B, dtype=jnp.int32) * N)[:, None, None]
    idx_flat = (idx + offs).reshape(GROWS)
    gathered = _make_sc_gather()(table, idx_flat)

    w1p = jnp.zeros((CPAD, 32), jnp.float32).at[:CIN].set(W1)
    pooled = _run_mlp(gathered, new_xyz.reshape(B * NPOINT, 3), w1p,
                      b1.reshape(1, 32), W2, b2.reshape(1, 32),
                      W3, b3.reshape(1, 64))
    new_features = pooled.reshape(B, NPOINT, 64).transpose(0, 2, 1)
    return new_xyz, new_features, fps_idx


# speedup vs baseline: 11.2889x; 1.0219x over previous
import functools

import jax
import jax.numpy as jnp
from jax import lax
from jax.experimental import pallas as pl
from jax.experimental.pallas import tpu as pltpu
from jax.experimental.pallas import tpu_sc as plsc

NPOINT = 1024
RADIUS = 0.2
NSAMPLE = 32
N = 16384
B = 4
CFEAT = 16
CIN = 3 + CFEAT
CPAD = 128
BIG_I32 = 1 << 30


def _fps_body(xr_ref, idx_out_ref, cx_out_ref, cy_out_ref, cz_out_ref):
    xs = xr_ref[...]
    x = xs[:, 0]
    y = xs[:, 1]
    z = xs[:, 2]
    row = lax.broadcasted_iota(jnp.int32, (B, 128, 128), 1)
    col = lax.broadcasted_iota(jnp.int32, (B, 128, 128), 2)
    flat = row * 128 + col
    prow = lax.broadcasted_iota(jnp.int32, (B, 8, 128), 1)
    pcol = lax.broadcasted_iota(jnp.int32, (B, 8, 128), 2)
    pos = prow * 128 + pcol

    def body(i, carry):
        dists, far, aidx, ax, ay, az = carry
        sel = pos == i
        aidx = jnp.where(sel, far, aidx)
        onehot = flat == far
        cx = jnp.sum(jnp.where(onehot, x, 0.0), axis=(1, 2), keepdims=True)
        cy = jnp.sum(jnp.where(onehot, y, 0.0), axis=(1, 2), keepdims=True)
        cz = jnp.sum(jnp.where(onehot, z, 0.0), axis=(1, 2), keepdims=True)
        ax = jnp.where(sel, cx, ax)
        ay = jnp.where(sel, cy, ay)
        az = jnp.where(sel, cz, az)
        d = (x - cx) ** 2 + (y - cy) ** 2 + (z - cz) ** 2
        dists = jnp.minimum(dists, d)
        m = jnp.max(dists, axis=(1, 2), keepdims=True)
        far = jnp.min(jnp.where(dists == m, flat, BIG_I32),
                      axis=(1, 2), keepdims=True)
        return dists, far, aidx, ax, ay, az

    dists0 = jnp.full((B, 128, 128), 1e10, dtype=jnp.float32)
    zi = jnp.zeros((B, 8, 128), dtype=jnp.int32)
    zf = jnp.zeros((B, 8, 128), dtype=jnp.float32)
    far0 = jnp.zeros((B, 1, 1), dtype=jnp.int32)
    _, _, aidx, ax, ay, az = lax.fori_loop(
        0, NPOINT, body, (dists0, far0, zi, zf, zf, zf))
    idx_out_ref[...] = aidx
    cx_out_ref[...] = ax
    cy_out_ref[...] = ay
    cz_out_ref[...] = az


def _run_fps(xr):
    sds = jax.ShapeDtypeStruct((B, 8, 128), jnp.float32)
    return pl.pallas_call(
        _fps_body,
        out_shape=[jax.ShapeDtypeStruct((B, 8, 128), jnp.int32),
                   sds, sds, sds],
    )(xr)


PTILE = 256


def _ballq_body(xt_ref, cents_ref, out_ref):
    xs = xt_ref[0]
    xx = xs[0:1, :]
    yy = xs[1:2, :]
    zz = xs[2:3, :]
    sx = xx * xx + yy * yy + zz * zz
    c = cents_ref[0]
    cx = c[:, 0:1]
    cy = c[:, 1:2]
    cz = c[:, 2:3]
    sc = cx * cx + cy * cy + cz * cz
    bf = lambda a: a.astype(jnp.bfloat16).astype(jnp.float32)
    dot = bf(cx) * bf(xx) + bf(cy) * bf(yy) + bf(cz) * bf(zz)
    d2 = sc + sx - 2.0 * dot
    colj = lax.broadcasted_iota(jnp.int32, (PTILE, N), 1)
    v0 = jnp.where(d2 <= RADIUS * RADIUS, colj, N)
    col32 = lax.broadcasted_iota(jnp.int32, (PTILE, NSAMPLE), 1)

    def body(s, carry):
        v, acc = carry
        m = jnp.min(v, axis=1, keepdims=True)
        rec = jnp.minimum(m, N)
        acc = jnp.where(col32 == s, rec, acc)
        v = jnp.where(v == m, BIG_I32, v)
        return v, acc

    acc0 = jnp.zeros((PTILE, NSAMPLE), dtype=jnp.int32)
    _, acc = lax.fori_loop(0, NSAMPLE, body, (v0, acc0))
    first = acc[:, 0:1]
    first = jnp.where(first == N, 0, first)
    acc = jnp.where(acc == N, first, acc)
    out_ref[0] = acc


def _run_ballq(xt, cents):
    return pl.pallas_call(
        _ballq_body,
        grid=(B, NPOINT // PTILE),
        in_specs=[
            pl.BlockSpec((1, 3, N), lambda b, p: (b, 0, 0)),
            pl.BlockSpec((1, PTILE, 3), lambda b, p: (b, p, 0)),
        ],
        out_specs=pl.BlockSpec((1, PTILE, NSAMPLE), lambda b, p: (b, p, 0)),
        out_shape=jax.ShapeDtypeStruct((B, NPOINT, NSAMPLE), jnp.int32),
    )(xt, cents)


GROWS = B * NPOINT * NSAMPLE
GCHUNK = 512


def _make_sc_gather():
    info = plsc.get_sparse_core_info()
    nw = info.num_cores * info.num_subcores
    b_per_w = GROWS // nw
    nchunks = b_per_w // GCHUNK
    mesh = plsc.VectorSubcoreMesh(core_axis_name="c", subcore_axis_name="s")

    @functools.partial(
        pl.kernel, mesh=mesh,
        out_type=jax.ShapeDtypeStruct((GROWS, CPAD), jnp.float32),
        scratch_types=[
            pltpu.VMEM((GCHUNK,), jnp.int32),
            pltpu.VMEM((GCHUNK, CPAD), jnp.float32),
            pltpu.SemaphoreType.DMA,
        ],
    )
    def sc_gather(table_hbm, idx_hbm, out_hbm, idx_v, rows_v, sem):
        wid = lax.axis_index("s") * info.num_cores + lax.axis_index("c")
        base = wid * b_per_w
        for k in range(nchunks):
            off = base + k * GCHUNK
            pltpu.sync_copy(idx_hbm.at[pl.ds(off, GCHUNK)], idx_v)
            pltpu.async_copy(table_hbm.at[idx_v], rows_v, sem).wait()
            pltpu.sync_copy(rows_v, out_hbm.at[pl.ds(off, GCHUNK)])

    return sc_gather


GT = 128
RT = GT * NSAMPLE


def _mlp_body(g_ref, c_ref, w1_ref, b1_ref, w2_ref, b2_ref, w3_ref, b3_ref,
              out_ref):
    g = g_ref[...]
    c = c_ref[...]
    w1 = w1_ref[...]
    cx = c[:, 0:1]
    cy = c[:, 1:2]
    cz = c[:, 2:3]
    base = cx * w1[0:1, :] + cy * w1[1:2, :] + cz * w1[2:3, :]
    base = jnp.broadcast_to(base[:, None, :], (GT, NSAMPLE, 32))
    base = base.reshape(RT, 32)
    h = jnp.dot(g, w1, preferred_element_type=jnp.float32)
    h = jnp.maximum(h + b1_ref[...] - base, 0.0)
    h = jnp.dot(h, w2_ref[...], preferred_element_type=jnp.float32)
    h = jnp.maximum(h + b2_ref[...], 0.0)
    h = jnp.dot(h, w3_ref[...], preferred_element_type=jnp.float32)
    h = jnp.maximum(h + b3_ref[...], 0.0)
    out_ref[...] = jnp.max(h.reshape(GT, NSAMPLE, 64), axis=1)


def _run_mlp(g, cflat, w1p, b1r, w2, b2r, w3, b3r):
    ngroups = B * NPOINT
    return pl.pallas_call(
        _mlp_body,
        grid=(ngroups // GT,),
        in_specs=[
            pl.BlockSpec((RT, CPAD), lambda i: (i, 0)),
            pl.BlockSpec((GT, 3), lambda i: (i, 0)),
            pl.BlockSpec((CPAD, 32), lambda i: (0, 0)),
            pl.BlockSpec((1, 32), lambda i: (0, 0)),
            pl.BlockSpec((32, 32), lambda i: (0, 0)),
            pl.BlockSpec((1, 32), lambda i: (0, 0)),
            pl.BlockSpec((32, 64), lambda i: (0, 0)),
            pl.BlockSpec((1, 64), lambda i: (0, 0)),
        ],
        out_specs=pl.BlockSpec((GT, 64), lambda i: (i, 0)),
        out_shape=jax.ShapeDtypeStruct((ngroups, 64), jnp.float32),
    )(g, cflat, w1p, b1r, w2, b2r, w3, b3r)


def kernel(xyz, features, W1, b1, W2, b2, W3, b3):
    xt = xyz.transpose(0, 2, 1)
    xr = xt.reshape(B, 3, 128, 128)

    fps_i, ax, ay, az = _run_fps(xr)
    fps_idx = fps_i.reshape(B, NPOINT)
    new_xyz = jnp.stack([ax.reshape(B, NPOINT), ay.reshape(B, NPOINT),
                         az.reshape(B, NPOINT)], axis=-1)

    idx = _run_ballq(xt, new_xyz)

    table = jnp.concatenate(
        [xyz, features,
         jnp.zeros((B, N, CPAD - CIN), dtype=jnp.float32)], axis=-1)
    table = table.reshape(B * N, CPAD)
    offs = (jnp.arange(B, dtype=jnp.int32) * N)[:, None, None]
    idx_flat = (idx + offs).reshape(GROWS)
    gathered = _make_sc_gather()(table, idx_flat)

    w1p = jnp.zeros((CPAD, 32), jnp.float32).at[:CIN].set(W1)
    pooled = _run_mlp(gathered, new_xyz.reshape(B * NPOINT, 3), w1p,
                      b1.reshape(1, 32), W2, b2.reshape(1, 32),
                      W3, b3.reshape(1, 64))
    new_features = pooled.reshape(B, NPOINT, 64).transpose(0, 2, 1)
    return new_xyz, new_features, fps_idx
